# baseline (speedup 1.0000x reference)
import functools

import jax
import jax.numpy as jnp
from jax import lax
from jax.experimental import pallas as pl
from jax.experimental.pallas import tpu as pltpu

N = 50000
NPG = 1000
B = 50
NHID = 128
NC = 2
NS = 16
LANES = 128
ACC_ROWS = 50048
ROWS_PER_SID = ACC_ROWS // NS
RB = 2
NRBUF = 4
BLK = 16
NCHUNK = 4
WCHUNK = NHID // NCHUNK


def _segsum_edges(table, srcp, dstp, zeros, width, cpt):
    from jax.experimental.pallas import tpu_sc as plsc

    mesh = plsc.VectorSubcoreMesh(core_axis_name="c", subcore_axis_name="s")
    nblk = cpt // BLK

    @functools.partial(
        pl.kernel,
        mesh=mesh,
        out_type=jax.ShapeDtypeStruct((NC, ACC_ROWS, width), jnp.float32),
        compiler_params=pltpu.CompilerParams(use_tc_tiling_on_sc=False),
        scratch_types=[
            pltpu.VMEM((BLK, LANES), jnp.int32),
            pltpu.VMEM((BLK, LANES), jnp.int32),
            pltpu.VMEM((NRBUF, LANES, width), jnp.float32),
            pltpu.VMEM_SHARED((ACC_ROWS, width), jnp.float32),
        ] + [pltpu.SemaphoreType.DMA] * (2 * NRBUF),
    )
    def seg_kernel(table_hbm, src_hbm, dst_hbm, zeros_hbm, out_hbm,
                   sidx, didx, rows, acc, *sems):
        gsem = sems[:NRBUF]
        ssem = sems[NRBUF:]
        cid = lax.axis_index("c")
        sid = lax.axis_index("s")
        wid = sid * NC + cid
        base_row = wid * cpt

        pltpu.sync_copy(
            zeros_hbm.at[pl.ds(sid * ROWS_PER_SID, ROWS_PER_SID)],
            acc.at[pl.ds(sid * ROWS_PER_SID, ROWS_PER_SID)])
        plsc.subcore_barrier()

        def blk_body(ib, carry):
            pltpu.sync_copy(src_hbm.at[pl.ds(base_row + ib * BLK, BLK)],
                            sidx)
            pltpu.sync_copy(dst_hbm.at[pl.ds(base_row + ib * BLK, BLK)],
                            didx)
            for j in range(RB):
                pltpu.async_copy(table_hbm.at[sidx.at[j]], rows.at[j],
                                 gsem[j])
            for j in range(BLK):
                b = j % NRBUF
                pltpu.make_async_copy(table_hbm.at[sidx.at[j]],
                                      rows.at[b], gsem[b]).wait()
                pltpu.async_copy(rows.at[b], acc.at[didx.at[j]],
                                 ssem[b], add=True)
                jn = j + RB
                if jn < BLK:
                    bn = jn % NRBUF
                    if j >= RB:
                        pltpu.make_async_copy(rows.at[bn],
                                              acc.at[didx.at[jn - NRBUF]],
                                              ssem[bn]).wait()
                    pltpu.async_copy(table_hbm.at[sidx.at[jn]],
                                     rows.at[bn], gsem[bn])
            for j in range(BLK - NRBUF, BLK):
                b = j % NRBUF
                pltpu.make_async_copy(rows.at[b], acc.at[didx.at[j]],
                                      ssem[b]).wait()
            return carry

        lax.fori_loop(0, nblk, blk_body, 0)
        plsc.subcore_barrier()
        pltpu.sync_copy(
            acc.at[pl.ds(sid * ROWS_PER_SID, ROWS_PER_SID)],
            out_hbm.at[cid, pl.ds(sid * ROWS_PER_SID, ROWS_PER_SID)])

    return seg_kernel(table, srcp, dstp, zeros)


SUBL = 8
LN = NPG // SUBL


def _topk_batch_body(k, s_ref, t_ref, sel_ref):
    s = s_ref[...]
    bits = lax.bitcast_convert_type(s, jnp.int32)
    mapped = bits ^ ((bits >> 31) & jnp.int32(0x7FFFFFFF))
    keyu = mapped.astype(jnp.uint32) ^ jnp.uint32(0x80000000)

    def count(mask):
        return jnp.sum(jnp.sum(mask.astype(jnp.int32), axis=2, keepdims=True),
                       axis=1, keepdims=True)

    prefix = jnp.zeros((B, 1, 1), jnp.uint32)
    for bit in range(31, -1, -1):
        cand = prefix | jnp.uint32(1 << bit)
        prefix = jnp.where(count(keyu >= cand) >= k, cand, prefix)
    gt = keyu > prefix
    eq = keyu == prefix
    need = k - count(gt)
    idx = (lax.broadcasted_iota(jnp.int32, (B, SUBL, LN), 1) * LN
           + lax.broadcasted_iota(jnp.int32, (B, SUBL, LN), 2))
    m = jnp.zeros((B, 1, 1), jnp.int32)
    for bit in range(9, -1, -1):
        cand = m | jnp.int32(1 << bit)
        m = jnp.where(count(jnp.logical_and(eq, idx < cand)) < need, cand, m)
    sel = jnp.logical_or(
        gt, jnp.logical_and(jnp.logical_and(eq, idx <= m), need > 0))
    sel_ref[...] = sel.astype(jnp.float32)
    t_ref[...] = jnp.where(sel, jnp.tanh(s), 0.0)


def _topk(score_col, k):
    s3 = score_col.reshape(B, SUBL, LN)
    t3, sel3 = pl.pallas_call(
        functools.partial(_topk_batch_body, k),
        in_specs=[pl.BlockSpec((B, SUBL, LN), lambda: (0, 0, 0))],
        out_specs=[pl.BlockSpec((B, SUBL, LN), lambda: (0, 0, 0))] * 2,
        out_shape=[jax.ShapeDtypeStruct((B, SUBL, LN), jnp.float32)] * 2,
    )(s3)
    return t3.reshape(N, 1), sel3.reshape(N, 1)


def _conv1_body(aggp_ref, x_ref, wrel_ref, wroot_ref, b_ref, p_ref,
                h_ref, s_ref):
    agg = aggp_ref[0] + aggp_ref[1]
    h = jnp.maximum(
        jnp.dot(agg, wrel_ref[...]) + jnp.dot(x_ref[...], wroot_ref[...])
        + b_ref[...], 0.0)
    h_ref[...] = h
    p = p_ref[...]
    s_ref[...] = jnp.dot(h, p) / jnp.sqrt(jnp.sum(p * p))


def _conv1(aggp, x16, wrel, wroot, b1, p1c):
    return pl.pallas_call(
        _conv1_body,
        grid=(B,),
        in_specs=[
            pl.BlockSpec((NC, NPG, 16), lambda i: (0, i, 0)),
            pl.BlockSpec((NPG, 16), lambda i: (i, 0)),
            pl.BlockSpec((16, NHID), lambda i: (0, 0)),
            pl.BlockSpec((16, NHID), lambda i: (0, 0)),
            pl.BlockSpec((1, NHID), lambda i: (0, 0)),
            pl.BlockSpec((NHID, 1), lambda i: (0, 0)),
        ],
        out_specs=[pl.BlockSpec((NPG, NHID), lambda i: (i, 0)),
                   pl.BlockSpec((NPG, 1), lambda i: (i, 0))],
        out_shape=[jax.ShapeDtypeStruct((N, NHID), jnp.float32),
                   jax.ShapeDtypeStruct((N, 1), jnp.float32)],
    )(aggp, x16, wrel, wroot, b1, p1c)


def _scale_split_body(h_ref, t_ref, hsf_ref, *c_refs):
    hs = h_ref[...] * t_ref[...]
    hsf_ref[...] = hs
    for c, ref in enumerate(c_refs):
        ref[...] = hs[:, c * WCHUNK:(c + 1) * WCHUNK]


def _scale_split(h, t):
    return pl.pallas_call(
        _scale_split_body,
        grid=(B,),
        in_specs=[
            pl.BlockSpec((NPG, NHID), lambda i: (i, 0)),
            pl.BlockSpec((NPG, 1), lambda i: (i, 0)),
        ],
        out_specs=[pl.BlockSpec((NPG, NHID), lambda i: (i, 0))]
        + [pl.BlockSpec((NPG, WCHUNK), lambda i: (i, 0))] * NCHUNK,
        out_shape=[jax.ShapeDtypeStruct((N, NHID), jnp.float32)]
        + [jax.ShapeDtypeStruct((N, WCHUNK), jnp.float32)] * NCHUNK,
    )(h, t)


def _conv2_body(*refs):
    part_refs = refs[:NCHUNK]
    (hsf_ref, sel_ref, wrel_ref, wroot_ref, b_ref, p_ref,
     g_ref, s_ref) = refs[NCHUNK:]
    agg2 = jnp.concatenate([r[0] + r[1] for r in part_refs], axis=1)
    g = jnp.maximum(
        jnp.dot(agg2, wrel_ref[...]) + jnp.dot(hsf_ref[...], wroot_ref[...])
        + b_ref[...], 0.0)
    g_ref[...] = g
    p = p_ref[...]
    s2 = jnp.dot(g, p) / jnp.sqrt(jnp.sum(p * p))
    s_ref[...] = jnp.where(sel_ref[...] > 0.0, s2, -jnp.inf)


def _conv2(parts, hsf, sel, wrel, wroot, b2, p2c):
    return pl.pallas_call(
        _conv2_body,
        grid=(B,),
        in_specs=[pl.BlockSpec((NC, NPG, WCHUNK), lambda i: (0, i, 0))] * NCHUNK
        + [
            pl.BlockSpec((NPG, NHID), lambda i: (i, 0)),
            pl.BlockSpec((NPG, 1), lambda i: (i, 0)),
            pl.BlockSpec((NHID, NHID), lambda i: (0, 0)),
            pl.BlockSpec((NHID, NHID), lambda i: (0, 0)),
            pl.BlockSpec((1, NHID), lambda i: (0, 0)),
            pl.BlockSpec((NHID, 1), lambda i: (0, 0)),
        ],
        out_specs=[
            pl.BlockSpec((NPG, NHID), lambda i: (i, 0)),
            pl.BlockSpec((NPG, 1), lambda i: (i, 0)),
        ],
        out_shape=[
            jax.ShapeDtypeStruct((N, NHID), jnp.float32),
            jax.ShapeDtypeStruct((N, 1), jnp.float32),
        ],
    )(*parts, hsf, sel, wrel, wroot, b2, p2c)


def _readout_body(k1, k2, hsf_ref, sel_ref, g_ref, t2_ref, sel2_ref,
                  wl1_ref, bl1_ref, wl2_ref, bl2_ref, wl3_ref, bl3_ref,
                  o_ref):
    hs = hsf_ref[...]
    ninf = jnp.float32(-jnp.inf)
    x1m = jnp.sum(hs, axis=0, keepdims=True) / k1
    x1x = jnp.max(jnp.where(sel_ref[...] > 0.0, hs, ninf),
                  axis=0, keepdims=True)
    gs = g_ref[...] * t2_ref[...]
    x2m = jnp.sum(gs, axis=0, keepdims=True) / k2
    x2x = jnp.max(jnp.where(sel2_ref[...] > 0.0, gs, ninf),
                  axis=0, keepdims=True)
    z = jnp.concatenate([x1m + x2m, x1x + x2x], axis=1)
    z = jnp.maximum(jnp.dot(z, wl1_ref[...]) + bl1_ref[...], 0.0)
    z = jnp.maximum(jnp.dot(z, wl2_ref[...]) + bl2_ref[...], 0.0)
    z = jnp.dot(z, wl3_ref[...]) + bl3_ref[...]
    m = jnp.max(z, axis=1, keepdims=True)
    lse = jnp.log(jnp.sum(jnp.exp(z - m), axis=1, keepdims=True))
    o_ref[...] = (z - m - lse)[None]


def _readout(hsf, sel, g, t2, sel2, wl1, bl1, wl2, bl2, wl3, bl3, k1, k2):
    return pl.pallas_call(
        functools.partial(_readout_body, float(k1), float(k2)),
        grid=(B,),
        in_specs=[
            pl.BlockSpec((NPG, NHID), lambda i: (i, 0)),
            pl.BlockSpec((NPG, 1), lambda i: (i, 0)),
            pl.BlockSpec((NPG, NHID), lambda i: (i, 0)),
            pl.BlockSpec((NPG, 1), lambda i: (i, 0)),
            pl.BlockSpec((NPG, 1), lambda i: (i, 0)),
            pl.BlockSpec((2 * NHID, NHID), lambda i: (0, 0)),
            pl.BlockSpec((1, NHID), lambda i: (0, 0)),
            pl.BlockSpec((NHID, 32), lambda i: (0, 0)),
            pl.BlockSpec((1, 32), lambda i: (0, 0)),
            pl.BlockSpec((32, 2), lambda i: (0, 0)),
            pl.BlockSpec((1, 2), lambda i: (0, 0)),
        ],
        out_specs=[pl.BlockSpec((1, 1, 2), lambda i: (i, 0, 0))],
        out_shape=[jax.ShapeDtypeStruct((B, 1, 2), jnp.float32)],
    )(hsf, sel, g, t2, sel2, wl1, bl1, wl2, bl2, wl3, bl3)[0]


def kernel(x, edge_index, batch, W1_rel, W1_root, b1, p1,
           W2_rel, W2_root, b2, p2, W_l1, b_l1, W_l2, b_l2, W_l3, b_l3):
    E = edge_index.shape[1]
    K1 = 800
    K2 = 640
    nw = NC * NS
    cpt = -(-E // (nw * LANES))
    cpt += (-cpt) % BLK
    e_pad = nw * cpt * LANES

    src = edge_index[0]
    dst = edge_index[1]
    pad = e_pad - E
    srcp = jnp.concatenate(
        [src, jnp.zeros((pad,), jnp.int32)]).reshape(nw * cpt, LANES)
    dstp = jnp.concatenate(
        [dst, jnp.full((pad,), N, jnp.int32)]).reshape(nw * cpt, LANES)

    x16 = jnp.pad(x, ((0, 0), (0, 2)))
    z16 = jnp.zeros((ACC_ROWS, 16), jnp.float32)
    zw = jnp.zeros((ACC_ROWS, WCHUNK), jnp.float32)
    wrel16 = jnp.pad(W1_rel, ((0, 2), (0, 0)))
    wroot16 = jnp.pad(W1_root, ((0, 2), (0, 0)))

    aggp = _segsum_edges(x16, srcp, dstp, z16, 16, cpt)
    h, score = _conv1(aggp, x16, wrel16, wroot16,
                      b1.reshape(1, NHID), p1.reshape(NHID, 1))
    t, sel = _topk(score, K1)
    hsf, *hs_chunks = _scale_split(h, t)
    parts = [_segsum_edges(hs, srcp, dstp, zw, WCHUNK, cpt)
             for hs in hs_chunks]
    g, score2m = _conv2(parts, hsf, sel, W2_rel, W2_root,
                        b2.reshape(1, NHID), p2.reshape(NHID, 1))
    t2, sel2 = _topk(score2m, K2)
    out = _readout(hsf, sel, g, t2, sel2,
                   W_l1, b_l1.reshape(1, NHID), W_l2, b_l2.reshape(1, 32),
                   W_l3, b_l3.reshape(1, 2), K1, K2)
    return out.reshape(B, 2)


# speedup vs baseline: 1.4208x; 1.4208x over previous
import functools

import jax
import jax.numpy as jnp
from jax import lax
from jax.experimental import pallas as pl
from jax.experimental.pallas import tpu as pltpu

N = 50000
NPG = 1000
B = 50
NHID = 128
NC = 2
NS = 16
LANES = 128
ACC_ROWS = 50048
ROWS_PER_SID = ACC_ROWS // NS
RB = 4
NCHUNK = 8
WCHUNK = NHID // NCHUNK


def _segsum_edges(table, srcp, dstp, zeros, width, cpt):
    from jax.experimental.pallas import tpu_sc as plsc

    mesh = plsc.VectorSubcoreMesh(core_axis_name="c", subcore_axis_name="s")
    nbuf = 2 * RB
    n_outer = cpt // nbuf

    @functools.partial(
        pl.kernel,
        mesh=mesh,
        out_type=jax.ShapeDtypeStruct((NC, ACC_ROWS, width), jnp.float32),
        compiler_params=pltpu.CompilerParams(use_tc_tiling_on_sc=False),
        scratch_types=[
            pltpu.VMEM((cpt, LANES), jnp.int32),
            pltpu.VMEM((cpt, LANES), jnp.int32),
            pltpu.VMEM((nbuf, LANES, width), jnp.float32),
            pltpu.VMEM_SHARED((ACC_ROWS, width), jnp.float32),
        ] + [pltpu.SemaphoreType.DMA] * (2 * nbuf),
    )
    def seg_kernel(table_hbm, src_hbm, dst_hbm, zeros_hbm, out_hbm,
                   sidx, didx, rows, acc, *sems):
        gsem = sems[:nbuf]
        ssem = sems[nbuf:]
        cid = lax.axis_index("c")
        sid = lax.axis_index("s")
        wid = sid * NC + cid

        pltpu.sync_copy(src_hbm.at[pl.ds(wid * cpt, cpt)], sidx)
        pltpu.sync_copy(dst_hbm.at[pl.ds(wid * cpt, cpt)], didx)
        for b in range(RB):
            pltpu.async_copy(table_hbm.at[sidx.at[b]], rows.at[b], gsem[b])
        pltpu.sync_copy(
            zeros_hbm.at[pl.ds(sid * ROWS_PER_SID, ROWS_PER_SID)],
            acc.at[pl.ds(sid * ROWS_PER_SID, ROWS_PER_SID)])
        plsc.subcore_barrier()

        def outer(g, carry):
            base = g * nbuf
            for b in range(nbuf):
                j = base + b
                pltpu.make_async_copy(table_hbm.at[sidx.at[j]],
                                      rows.at[b], gsem[b]).wait()
                pltpu.async_copy(rows.at[b], acc.at[didx.at[j]],
                                 ssem[b], add=True)
                bn = (b + RB) % nbuf
                jn = j + RB

                @pl.when(jn < cpt)
                def _():
                    @pl.when(j >= RB)
                    def _():
                        pltpu.make_async_copy(
                            rows.at[bn], acc.at[didx.at[jn - nbuf]],
                            ssem[bn]).wait()
                    pltpu.async_copy(table_hbm.at[sidx.at[jn]],
                                     rows.at[bn], gsem[bn])
            return carry

        lax.fori_loop(0, n_outer, outer, 0)
        for b in range(nbuf):
            j = cpt - nbuf + b
            pltpu.make_async_copy(rows.at[b], acc.at[didx.at[j]],
                                  ssem[b]).wait()
        plsc.subcore_barrier()
        pltpu.sync_copy(
            acc.at[pl.ds(sid * ROWS_PER_SID, ROWS_PER_SID)],
            out_hbm.at[cid, pl.ds(sid * ROWS_PER_SID, ROWS_PER_SID)])

    return seg_kernel(table, srcp, dstp, zeros)


SUBL = 8
LN = NPG // SUBL


def _topk_batch_body(k, s_ref, t_ref, sel_ref):
    s = s_ref[...]
    bits = lax.bitcast_convert_type(s, jnp.int32)
    mapped = bits ^ ((bits >> 31) & jnp.int32(0x7FFFFFFF))
    keyu = mapped.astype(jnp.uint32) ^ jnp.uint32(0x80000000)

    def count(mask):
        return jnp.sum(jnp.sum(mask.astype(jnp.int32), axis=2, keepdims=True),
                       axis=1, keepdims=True)

    prefix = jnp.zeros((B, 1, 1), jnp.uint32)
    for bit in range(31, -1, -1):
        cand = prefix | jnp.uint32(1 << bit)
        prefix = jnp.where(count(keyu >= cand) >= k, cand, prefix)
    gt = keyu > prefix
    eq = keyu == prefix
    need = k - count(gt)
    idx = (lax.broadcasted_iota(jnp.int32, (B, SUBL, LN), 1) * LN
           + lax.broadcasted_iota(jnp.int32, (B, SUBL, LN), 2))
    m = jnp.zeros((B, 1, 1), jnp.int32)
    for bit in range(9, -1, -1):
        cand = m | jnp.int32(1 << bit)
        m = jnp.where(count(jnp.logical_and(eq, idx < cand)) < need, cand, m)
    sel = jnp.logical_or(
        gt, jnp.logical_and(jnp.logical_and(eq, idx <= m), need > 0))
    sel_ref[...] = sel.astype(jnp.float32)
    t_ref[...] = jnp.where(sel, jnp.tanh(s), 0.0)


def _topk(score_col, k):
    s3 = score_col.reshape(B, SUBL, LN)
    t3, sel3 = pl.pallas_call(
        functools.partial(_topk_batch_body, k),
        in_specs=[pl.BlockSpec((B, SUBL, LN), lambda: (0, 0, 0))],
        out_specs=[pl.BlockSpec((B, SUBL, LN), lambda: (0, 0, 0))] * 2,
        out_shape=[jax.ShapeDtypeStruct((B, SUBL, LN), jnp.float32)] * 2,
    )(s3)
    return t3.reshape(N, 1), sel3.reshape(N, 1)


def _conv1_body(aggp_ref, x_ref, wrel_ref, wroot_ref, b_ref, p_ref,
                h_ref, s_ref):
    agg = aggp_ref[0] + aggp_ref[1]
    h = jnp.maximum(
        jnp.dot(agg, wrel_ref[...]) + jnp.dot(x_ref[...], wroot_ref[...])
        + b_ref[...], 0.0)
    h_ref[...] = h
    p = p_ref[...]
    s_ref[...] = jnp.dot(h, p) / jnp.sqrt(jnp.sum(p * p))


def _conv1(aggp, x16, wrel, wroot, b1, p1c):
    return pl.pallas_call(
        _conv1_body,
        grid=(B,),
        in_specs=[
            pl.BlockSpec((NC, NPG, 16), lambda i: (0, i, 0)),
            pl.BlockSpec((NPG, 16), lambda i: (i, 0)),
            pl.BlockSpec((16, NHID), lambda i: (0, 0)),
            pl.BlockSpec((16, NHID), lambda i: (0, 0)),
            pl.BlockSpec((1, NHID), lambda i: (0, 0)),
            pl.BlockSpec((NHID, 1), lambda i: (0, 0)),
        ],
        out_specs=[pl.BlockSpec((NPG, NHID), lambda i: (i, 0)),
                   pl.BlockSpec((NPG, 1), lambda i: (i, 0))],
        out_shape=[jax.ShapeDtypeStruct((N, NHID), jnp.float32),
                   jax.ShapeDtypeStruct((N, 1), jnp.float32)],
    )(aggp, x16, wrel, wroot, b1, p1c)


def _scale_split_body(h_ref, t_ref, hsf_ref, *c_refs):
    hs = h_ref[...] * t_ref[...]
    hsf_ref[...] = hs
    for c, ref in enumerate(c_refs):
        ref[...] = hs[:, c * WCHUNK:(c + 1) * WCHUNK]


def _scale_split(h, t):
    return pl.pallas_call(
        _scale_split_body,
        grid=(B,),
        in_specs=[
            pl.BlockSpec((NPG, NHID), lambda i: (i, 0)),
            pl.BlockSpec((NPG, 1), lambda i: (i, 0)),
        ],
        out_specs=[pl.BlockSpec((NPG, NHID), lambda i: (i, 0))]
        + [pl.BlockSpec((NPG, WCHUNK), lambda i: (i, 0))] * NCHUNK,
        out_shape=[jax.ShapeDtypeStruct((N, NHID), jnp.float32)]
        + [jax.ShapeDtypeStruct((N, WCHUNK), jnp.float32)] * NCHUNK,
    )(h, t)


def _conv2_body(*refs):
    part_refs = refs[:NCHUNK]
    (hsf_ref, sel_ref, wrel_ref, wroot_ref, b_ref, p_ref,
     g_ref, s_ref) = refs[NCHUNK:]
    agg2 = jnp.concatenate([r[0] + r[1] for r in part_refs], axis=1)
    g = jnp.maximum(
        jnp.dot(agg2, wrel_ref[...]) + jnp.dot(hsf_ref[...], wroot_ref[...])
        + b_ref[...], 0.0)
    g_ref[...] = g
    p = p_ref[...]
    s2 = jnp.dot(g, p) / jnp.sqrt(jnp.sum(p * p))
    s_ref[...] = jnp.where(sel_ref[...] > 0.0, s2, -jnp.inf)


def _conv2(parts, hsf, sel, wrel, wroot, b2, p2c):
    return pl.pallas_call(
        _conv2_body,
        grid=(B,),
        in_specs=[pl.BlockSpec((NC, NPG, WCHUNK), lambda i: (0, i, 0))] * NCHUNK
        + [
            pl.BlockSpec((NPG, NHID), lambda i: (i, 0)),
            pl.BlockSpec((NPG, 1), lambda i: (i, 0)),
            pl.BlockSpec((NHID, NHID), lambda i: (0, 0)),
            pl.BlockSpec((NHID, NHID), lambda i: (0, 0)),
            pl.BlockSpec((1, NHID), lambda i: (0, 0)),
            pl.BlockSpec((NHID, 1), lambda i: (0, 0)),
        ],
        out_specs=[
            pl.BlockSpec((NPG, NHID), lambda i: (i, 0)),
            pl.BlockSpec((NPG, 1), lambda i: (i, 0)),
        ],
        out_shape=[
            jax.ShapeDtypeStruct((N, NHID), jnp.float32),
            jax.ShapeDtypeStruct((N, 1), jnp.float32),
        ],
    )(*parts, hsf, sel, wrel, wroot, b2, p2c)


def _readout_body(k1, k2, hsf_ref, sel_ref, g_ref, t2_ref, sel2_ref,
                  wl1_ref, bl1_ref, wl2_ref, bl2_ref, wl3_ref, bl3_ref,
                  o_ref):
    hs = hsf_ref[...]
    ninf = jnp.float32(-jnp.inf)
    x1m = jnp.sum(hs, axis=0, keepdims=True) / k1
    x1x = jnp.max(jnp.where(sel_ref[...] > 0.0, hs, ninf),
                  axis=0, keepdims=True)
    gs = g_ref[...] * t2_ref[...]
    x2m = jnp.sum(gs, axis=0, keepdims=True) / k2
    x2x = jnp.max(jnp.where(sel2_ref[...] > 0.0, gs, ninf),
                  axis=0, keepdims=True)
    z = jnp.concatenate([x1m + x2m, x1x + x2x], axis=1)
    z = jnp.maximum(jnp.dot(z, wl1_ref[...]) + bl1_ref[...], 0.0)
    z = jnp.maximum(jnp.dot(z, wl2_ref[...]) + bl2_ref[...], 0.0)
    z = jnp.dot(z, wl3_ref[...]) + bl3_ref[...]
    m = jnp.max(z, axis=1, keepdims=True)
    lse = jnp.log(jnp.sum(jnp.exp(z - m), axis=1, keepdims=True))
    o_ref[...] = (z - m - lse)[None]


def _readout(hsf, sel, g, t2, sel2, wl1, bl1, wl2, bl2, wl3, bl3, k1, k2):
    return pl.pallas_call(
        functools.partial(_readout_body, float(k1), float(k2)),
        grid=(B,),
        in_specs=[
            pl.BlockSpec((NPG, NHID), lambda i: (i, 0)),
            pl.BlockSpec((NPG, 1), lambda i: (i, 0)),
            pl.BlockSpec((NPG, NHID), lambda i: (i, 0)),
            pl.BlockSpec((NPG, 1), lambda i: (i, 0)),
            pl.BlockSpec((NPG, 1), lambda i: (i, 0)),
            pl.BlockSpec((2 * NHID, NHID), lambda i: (0, 0)),
            pl.BlockSpec((1, NHID), lambda i: (0, 0)),
            pl.BlockSpec((NHID, 32), lambda i: (0, 0)),
            pl.BlockSpec((1, 32), lambda i: (0, 0)),
            pl.BlockSpec((32, 2), lambda i: (0, 0)),
            pl.BlockSpec((1, 2), lambda i: (0, 0)),
        ],
        out_specs=[pl.BlockSpec((1, 1, 2), lambda i: (i, 0, 0))],
        out_shape=[jax.ShapeDtypeStruct((B, 1, 2), jnp.float32)],
    )(hsf, sel, g, t2, sel2, wl1, bl1, wl2, bl2, wl3, bl3)[0]


def kernel(x, edge_index, batch, W1_rel, W1_root, b1, p1,
           W2_rel, W2_root, b2, p2, W_l1, b_l1, W_l2, b_l2, W_l3, b_l3):
    E = edge_index.shape[1]
    K1 = 800
    K2 = 640
    nw = NC * NS
    cpt = -(-E // (nw * LANES))
    cpt += (-cpt) % (2 * RB)
    e_pad = nw * cpt * LANES

    src = edge_index[0]
    dst = edge_index[1]
    pad = e_pad - E
    srcp = jnp.concatenate(
        [src, jnp.zeros((pad,), jnp.int32)]).reshape(nw * cpt, LANES)
    dstp = jnp.concatenate(
        [dst, jnp.full((pad,), N, jnp.int32)]).reshape(nw * cpt, LANES)

    x16 = jnp.pad(x, ((0, 0), (0, 2)))
    z16 = jnp.zeros((ACC_ROWS, 16), jnp.float32)
    wrel16 = jnp.pad(W1_rel, ((0, 2), (0, 0)))
    wroot16 = jnp.pad(W1_root, ((0, 2), (0, 0)))

    aggp = _segsum_edges(x16, srcp, dstp, z16, 16, cpt)
    h, score = _conv1(aggp, x16, wrel16, wroot16,
                      b1.reshape(1, NHID), p1.reshape(NHID, 1))
    t, sel = _topk(score, K1)
    hsf, *hs_chunks = _scale_split(h, t)
    parts = [_segsum_edges(hs, srcp, dstp, z16, WCHUNK, cpt)
             for hs in hs_chunks]
    g, score2m = _conv2(parts, hsf, sel, W2_rel, W2_root,
                        b2.reshape(1, NHID), p2.reshape(NHID, 1))
    t2, sel2 = _topk(score2m, K2)
    out = _readout(hsf, sel, g, t2, sel2,
                   W_l1, b_l1.reshape(1, NHID), W_l2, b_l2.reshape(1, 32),
                   W_l3, b_l3.reshape(1, 2), K1, K2)
    return out.reshape(B, 2)
